# baseline (speedup 1.0000x reference)
import jax
import jax.numpy as jnp
from jax.experimental import pallas as pl
from jax.experimental.pallas import tpu as pltpu

_BM = 256


def _body(a1_ref, a2_ref, e_ref, o1_ref, o2_ref):
    o1_ref[...] = a1_ref[:, :64]
    o2_ref[...] = a2_ref[:, :64]


def kernel(matrix_parents, Epsilon):
    M, K = matrix_parents.shape
    _, N = Epsilon.shape
    H = M // 2
    nsteps = H // _BM
    o1, o2 = pl.pallas_call(
        _body,
        grid=(nsteps,),
        in_specs=[
            pl.BlockSpec((_BM, K), lambda i: (i, 0)),
            pl.BlockSpec((_BM, K), lambda i, _n=nsteps: (i + _n, 0)),
            pl.BlockSpec((K, N), lambda i: (0, 0)),
        ],
        out_specs=[
            pl.BlockSpec((_BM, N), lambda i: (i, 0)),
            pl.BlockSpec((_BM, N), lambda i: (i, 0)),
        ],
        out_shape=[
            jax.ShapeDtypeStruct((H, N), jnp.float32),
            jax.ShapeDtypeStruct((H, N), jnp.float32),
        ],
        compiler_params=pltpu.CompilerParams(
            dimension_semantics=("arbitrary",),
        ),
    )(matrix_parents, matrix_parents, Epsilon)
    return jnp.concatenate([o1, o2], axis=0)


# speedup vs baseline: 1.0030x; 1.0030x over previous
import jax
import jax.numpy as jnp
from jax.experimental import pallas as pl
from jax.experimental.pallas import tpu as pltpu

_BM = 256


def _body(a_ref, e_ref, o_ref, ebf_ref):
    @pl.when(pl.program_id(0) == 0)
    def _():
        ebf_ref[...] = e_ref[...].astype(jnp.bfloat16)

    o_ref[...] = jax.lax.dot_general(
        a_ref[...].astype(jnp.bfloat16), ebf_ref[...],
        dimension_numbers=(((1,), (0,)), ((), ())),
        preferred_element_type=jnp.float32,
    )


def kernel(matrix_parents, Epsilon):
    M, K = matrix_parents.shape
    _, N = Epsilon.shape
    return pl.pallas_call(
        _body,
        grid=(M // _BM,),
        in_specs=[
            pl.BlockSpec((_BM, K), lambda i: (i, 0)),
            pl.BlockSpec((K, N), lambda i: (0, 0)),
        ],
        out_specs=pl.BlockSpec((_BM, N), lambda i: (i, 0)),
        out_shape=jax.ShapeDtypeStruct((M, N), jnp.float32),
        scratch_shapes=[pltpu.VMEM((K, N), jnp.bfloat16)],
        compiler_params=pltpu.CompilerParams(
            dimension_semantics=("arbitrary",),
            disable_bounds_checks=True,
        ),
    )(matrix_parents, Epsilon)


# speedup vs baseline: 1.0081x; 1.0050x over previous
import jax
import jax.numpy as jnp
from jax.experimental import pallas as pl
from jax.experimental.pallas import tpu as pltpu

_BM = 256


def _body(a_ref, e_ref, o_ref, ebf_ref):
    @pl.when(pl.program_id(0) == 0)
    def _():
        ebf_ref[...] = e_ref[...].astype(jnp.bfloat16)

    o_ref[...] = jax.lax.dot_general(
        a_ref[...], ebf_ref[...],
        dimension_numbers=(((1,), (0,)), ((), ())),
        preferred_element_type=jnp.float32,
    )


def kernel(matrix_parents, Epsilon):
    M, K = matrix_parents.shape
    _, N = Epsilon.shape
    return pl.pallas_call(
        _body,
        grid=(M // _BM,),
        in_specs=[
            pl.BlockSpec((_BM, K), lambda i: (i, 0)),
            pl.BlockSpec((K, N), lambda i: (0, 0)),
        ],
        out_specs=pl.BlockSpec((_BM, N), lambda i: (i, 0)),
        out_shape=jax.ShapeDtypeStruct((M, N), jnp.float32),
        scratch_shapes=[pltpu.VMEM((K, N), jnp.bfloat16)],
        compiler_params=pltpu.CompilerParams(
            dimension_semantics=("arbitrary",),
            disable_bounds_checks=True,
        ),
    )(matrix_parents, Epsilon)
